# baseline (speedup 1.0000x reference)
import jax
import jax.numpy as jnp
from jax.experimental import pallas as pl

B = 16
N = 16384
S = 512
K = 16

_HIGH = jax.lax.Precision.HIGHEST


def _fps_kernel(x_ref, out_ref):
    x0 = x_ref[0]
    x1 = x_ref[1]
    x2 = x_ref[2]
    iota_n = jax.lax.broadcasted_iota(jnp.int32, (B, N), 1)
    iota_s = jax.lax.broadcasted_iota(jnp.int32, (B, S), 1)

    def body(i, carry):
        dists, far, p0, p1, p2 = carry
        mask = iota_n == far
        c0 = jnp.sum(jnp.where(mask, x0, 0.0), axis=1, keepdims=True)
        c1 = jnp.sum(jnp.where(mask, x1, 0.0), axis=1, keepdims=True)
        c2 = jnp.sum(jnp.where(mask, x2, 0.0), axis=1, keepdims=True)
        rec = iota_s == i
        p0 = jnp.where(rec, c0, p0)
        p1 = jnp.where(rec, c1, p1)
        p2 = jnp.where(rec, c2, p2)
        d = (x0 - c0) ** 2 + (x1 - c1) ** 2 + (x2 - c2) ** 2
        dists = jnp.minimum(dists, d)
        dmax = jnp.max(dists, axis=1, keepdims=True)
        far = jnp.min(
            jnp.where(dists == dmax, iota_n, N), axis=1, keepdims=True
        )
        return dists, far, p0, p1, p2

    dists0 = jnp.full((B, N), 1e10, dtype=jnp.float32)
    far0 = jnp.zeros((B, 1), dtype=jnp.int32)
    z = jnp.zeros((B, S), dtype=jnp.float32)
    _, _, p0, p1, p2 = jax.lax.fori_loop(0, S, body, (dists0, far0, z, z, z))
    out_ref[0] = p0
    out_ref[1] = p1
    out_ref[2] = p2


def _knn_adjacency(neg):
    iota_r = jax.lax.broadcasted_iota(jnp.int32, (S, S), 0)

    def body(j, carry):
        neg, A = carry
        cmax = jnp.max(neg, axis=0, keepdims=True)
        r = jnp.min(
            jnp.where(neg == cmax, iota_r, S), axis=0, keepdims=True
        )
        hit = iota_r == r
        neg = jnp.where(hit, -1e30, neg)
        A = jnp.where(hit, 1.0, A)
        return neg, A

    _, A = jax.lax.fori_loop(
        0, K, body, (neg, jnp.zeros((S, S), jnp.float32))
    )
    return A


def _neg_dist(fT):
    G = jax.lax.dot_general(
        fT, fT, (((1,), (1,)), ((), ())),
        precision=_HIGH, preferred_element_type=jnp.float32,
    )
    xx = jnp.sum(fT * fT, axis=1, keepdims=True)
    inner = -2.0 * G
    return (-xx - inner) - jnp.transpose(xx)


def _mmT(a, w):
    return jax.lax.dot_general(
        a, w, (((1,), (1,)), ((), ())),
        precision=_HIGH, preferred_element_type=jnp.float32,
    )


def _stage(fT, peT, Wq, Wk, Wv, C, D):
    WqL, WqR = Wq[:, :C], Wq[:, C:]
    WkL, WkR = Wk[:, :C], Wk[:, C:]
    WvL, WvR = Wv[:, :C], Wv[:, C:]
    DT = _mmT(fT, WqL - WkL)
    AvT = _mmT(fT, WvL)
    FT = _mmT(fT, WvR - WvL) + peT

    A = _knn_adjacency(_neg_dist(fT))

    expD = jnp.exp(DT - jnp.max(DT, axis=0, keepdims=True))
    cat = jnp.concatenate([expD, expD * AvT], axis=1)
    R = jax.lax.dot_general(
        A, cat, (((0,), (0,)), ((), ())),
        precision=_HIGH, preferred_element_type=jnp.float32,
    )
    return R[:, D:] / R[:, :D] + FT


def _stages_kernel(pT_ref, wq1, wk1, wv1, wp1, wq2, wk2, wv2, wp2,
                   wq3, wk3, wv3, wp3, out_ref):
    pT = pT_ref[0, :, 0:3]

    pe1T = _mmT(pT, wp1[...])
    x1T = _stage(pT, pe1T, wq1[...], wk1[...], wv1[...], 3, 64)
    pe2T = _mmT(pT, wp2[...])
    x2T = _stage(x1T, pe2T, wq2[...], wk2[...], wv2[...], 64, 64)
    pe3T = _mmT(pT, wp3[...])
    x3T = _stage(x2T, pe3T, wq3[...], wk3[...], wv3[...], 64, 128)

    xcT = jnp.concatenate([x1T, x2T, x3T], axis=1)
    pmax = jnp.max(xcT, axis=0, keepdims=True)
    pmean = jnp.mean(xcT, axis=0, keepdims=True)
    out_ref[0] = jnp.concatenate([pmax, pmean], axis=1)


@jax.jit
def kernel(x, Wq1, Wk1, Wv1, Wp1, Wq2, Wk2, Wv2, Wp2, Wq3, Wk3, Wv3, Wp3):
    xT = jnp.transpose(x, (2, 0, 1))
    partial3 = pl.pallas_call(
        _fps_kernel,
        out_shape=jax.ShapeDtypeStruct((3, B, S), jnp.float32),
    )(xT)

    pT = jnp.transpose(partial3, (1, 2, 0))
    pT = jnp.pad(pT, ((0, 0), (0, 0), (0, 5)))

    ws = [Wq1, Wk1, Wv1, Wp1, Wq2, Wk2, Wv2, Wp2, Wq3, Wk3, Wv3, Wp3]
    out = pl.pallas_call(
        _stages_kernel,
        grid=(B,),
        in_specs=[pl.BlockSpec((1, S, 8), lambda b: (b, 0, 0))]
        + [pl.BlockSpec(w.shape, lambda b, nd=w.ndim: (0,) * nd) for w in ws],
        out_specs=pl.BlockSpec((1, 1, 512), lambda b: (b, 0, 0)),
        out_shape=jax.ShapeDtypeStruct((B, 1, 512), jnp.float32),
    )(pT, *ws)
    return out.reshape(B, 512)
